# baseline (speedup 1.0000x reference)
import functools

import numpy as np
import jax
import jax.numpy as jnp
from jax import lax
from jax.experimental import pallas as pl
from jax.experimental.pallas import tpu as pltpu
from jax.experimental.pallas import tpu_sc as plsc

_B, _NV, _NF = 2, 20000, 25000
_NFP = 25088
_NW = 32
_FPW = _NFP // _NW
_FB = 512
_NB = _NFP // _FB
_PI = float(np.pi)

_FEATS = ([(64 * k, 64) for k in range(9)]
          + [(576 + 16 * a, 16) for a in range(3)]
          + [(624, 16)]
          + [(640 + 64 * a, 64) for a in range(3)]
          + [(832, 16)])
_EMANGLE_OFF = 848
_EMFREQ_OFF = 1040


def _dis(t, lo, hi, num):
    t2 = (t - lo) / (hi - lo) * num - 0.5
    return jnp.clip(jnp.round(t2), 0, num - 1).astype(jnp.int32)


@functools.cache
def _make_sc_gather():
    mesh = plsc.VectorSubcoreMesh(core_axis_name="c", subcore_axis_name="s")

    @functools.partial(
        pl.kernel,
        out_type=jax.ShapeDtypeStruct((_B * 9 * _NFP,), jnp.float32),
        scratch_types=[
            pltpu.VMEM((3 * _NV,), jnp.float32),
            pltpu.VMEM((3 * _FPW,), jnp.int32),
            pltpu.VMEM((9 * _FPW,), jnp.float32),
        ],
        compiler_params=pltpu.CompilerParams(needs_layout_passes=False),
        mesh=mesh,
    )
    def _sc_body(vt_hbm, faces_hbm, fc_hbm, vbuf, ibuf, obuf):
        wid = lax.axis_index("s") * 2 + lax.axis_index("c")
        base = wid * _FPW
        lane3 = jnp.arange(16, dtype=jnp.int32) * 3
        for b in range(_B):
            pltpu.sync_copy(vt_hbm.at[pl.ds(b * 3 * _NV, 3 * _NV)], vbuf)
            pltpu.sync_copy(faces_hbm.at[pl.ds((b * _NFP + base) * 3, 3 * _FPW)],
                            ibuf)

            def body(g, carry):
                for s in range(3):
                    vidx = plsc.load_gather(ibuf, [g * 48 + s + lane3])
                    for c in range(3):
                        vals = plsc.load_gather(vbuf, [vidx * 3 + c])
                        obuf[pl.ds((3 * s + c) * _FPW + g * 16, 16)] = vals
                return carry

            lax.fori_loop(0, _FPW // 16, body, 0)
            for p in range(9):
                pltpu.sync_copy(obuf.at[pl.ds(p * _FPW, _FPW)],
                                fc_hbm.at[pl.ds((b * 9 + p) * _NFP + base, _FPW)])

    return _sc_body


def _sc_gather(vt, faces_t):
    fc_flat = _make_sc_gather()(vt.reshape(-1), faces_t.reshape(-1))
    return fc_flat.reshape(_B, 9, _NFP)


def _prep_body(coor_ref, angle_ref, area_ref, normal_ref, emno_ref,
               emang_ref, emfreq_ref, w_ref, bproj_ref, bins_ref,
               tt_ref, bias_ref):
    embs = [coor_ref] * 9 + [angle_ref] * 3 + [area_ref] + [normal_ref] * 3 + [emno_ref]
    for j, (off, c) in enumerate(_FEATS):
        w = w_ref[off:off + c, :]
        e = embs[j][:, :]
        tjt = lax.dot_general(w, e, (((0,), (1,)), ((), ())),
                              preferred_element_type=jnp.float32)
        tt_ref[:, 128 * j:128 * (j + 1)] = tjt.astype(jnp.bfloat16)

    bias = jnp.broadcast_to(bproj_ref[:, :], (8, 128))
    si = lax.broadcasted_iota(jnp.int32, (8, 128), 0)
    li = lax.broadcasted_iota(jnp.int32, (8, 128), 1)
    for a in range(3):
        c0 = bins_ref[0, a]
        c1 = bins_ref[1, a]
        tgt = jnp.where(si == 0, c0, jnp.where(si == 1, c1, -1))
        oh = (li == tgt).astype(jnp.float32)
        wa = w_ref[_EMANGLE_OFF + 64 * a:_EMANGLE_OFF + 64 * a + 64, :]
        embw = lax.dot_general(emang_ref[:, :], wa, (((1,), (0,)), ((), ())),
                               preferred_element_type=jnp.float32)
        bias = bias + lax.dot_general(oh, embw, (((1,), (0,)), ((), ())),
                                      preferred_element_type=jnp.float32)
    sif = lax.broadcasted_iota(jnp.int32, (8, 512), 0)
    lif = lax.broadcasted_iota(jnp.int32, (8, 512), 1)
    f0 = bins_ref[0, 3]
    f1 = bins_ref[1, 3]
    tgtf = jnp.where(sif == 0, f0, jnp.where(sif == 1, f1, -1))
    ohf = (lif == tgtf).astype(jnp.float32)
    embwf = lax.dot_general(emfreq_ref[:, :], w_ref[_EMFREQ_OFF:, :],
                            (((1,), (0,)), ((), ())),
                            preferred_element_type=jnp.float32)
    bias = bias + lax.dot_general(ohf, embwf, (((1,), (0,)), ((), ())),
                                  preferred_element_type=jnp.float32)
    bias_ref[:, :] = bias


def _prep_call(embeds, w, bprojT, em_bins):
    n_in = len(embeds) + 2
    in_specs = [pl.BlockSpec(x.shape, lambda i, _r=x.ndim: (0,) * _r)
                for x in (*embeds, w, bprojT)]
    in_specs.append(pl.BlockSpec(memory_space=pltpu.SMEM))
    return pl.pallas_call(
        _prep_body,
        grid=(1,),
        in_specs=in_specs,
        out_specs=[pl.BlockSpec((128, 2176), lambda i: (0, 0)),
                   pl.BlockSpec((8, 128), lambda i: (0, 0))],
        out_shape=[jax.ShapeDtypeStruct((128, 2176), jnp.bfloat16),
                   jax.ShapeDtypeStruct((8, 128), jnp.float32)],
    )(*embeds, w, bprojT, em_bins)


def _acos(x):
    ax = jnp.abs(x)
    p = jnp.float32(-0.0012624911)
    for coef in (0.0066700901, -0.0170881256, 0.0308918810, -0.0501743046,
                 0.0889789874, -0.2145988016, 1.5707963050):
        p = p * ax + jnp.float32(coef)
    r = jnp.sqrt(jnp.maximum(1.0 - ax, 0.0)) * p
    return jnp.where(x < 0, _PI - r, r)


def _main_body(vecn_ref, fc_ref, tt_ref, bias_ref, out_ref, oh_ref):
    b = pl.program_id(0)
    v = [[fc_ref[0, 3 * s + c:3 * s + c + 1, :] for c in range(3)] for s in range(3)]
    e = [[v[s][c] - v[s - 1][c] for c in range(3)] for s in range(3)]
    nh = []
    for s in range(3):
        n2 = e[s][0] * e[s][0] + e[s][1] * e[s][1] + e[s][2] * e[s][2]
        den = jnp.maximum(jnp.sqrt(n2), 1e-12)
        nh.append([e[s][c] / den for c in range(3)])

    idxs = []
    for s in range(3):
        for c in range(3):
            idxs.append(_dis(v[s][c], -1.0, 1.0, 128))
    for c in range(3):
        nd = -(nh[0][c] * nh[0][c - 1] + nh[1][c] * nh[1][c - 1]
               + nh[2][c] * nh[2][c - 1])
        ang = _acos(jnp.clip(nd, -1.0 + 1e-5, 1.0 - 1e-5))
        idxs.append(_dis(ang, 0.0, _PI, 128))
    crx = e[0][1] * e[1][2] - e[0][2] * e[1][1]
    cry = e[0][2] * e[1][0] - e[0][0] * e[1][2]
    crz = e[0][0] * e[1][1] - e[0][1] * e[1][0]
    crn = jnp.sqrt(crx * crx + cry * cry + crz * crz)
    dn = jnp.maximum(crn, 1e-12)
    nr = [crx / dn, cry / dn, crz / dn]
    idxs.append(_dis(crn * 0.5, 0.0, 4.0, 128))
    for c in range(3):
        idxs.append(_dis(nr[c], -1.0, 1.0, 128))
    nn = jnp.sqrt(nr[0] * nr[0] + nr[1] * nr[1] + nr[2] * nr[2])
    dnn = jnp.maximum(nn, 1e-12)
    vx = vecn_ref[b, 0]
    vy = vecn_ref[b, 1]
    vz = vecn_ref[b, 2]
    nd2 = -((nr[0] / dnn) * vx + (nr[1] / dnn) * vy + (nr[2] / dnn) * vz)
    emno = _acos(jnp.clip(nd2, -1.0 + 1e-5, 1.0 - 1e-5))
    idxs.append(_dis(emno, 0.0, _PI, 128))

    riota = lax.broadcasted_iota(jnp.int32, (128, _FB), 0)
    for j in range(17):
        oh_ref[128 * j:128 * (j + 1), :] = (riota == idxs[j]).astype(jnp.bfloat16)
    acc = lax.dot_general(oh_ref[:, :], tt_ref[:, :], (((0,), (1,)), ((), ())),
                          preferred_element_type=jnp.float32)
    out_ref[0, :, :] = acc + bias_ref[0, :, :]


def _main_call(vecn, fc, tt, bias3):
    return pl.pallas_call(
        _main_body,
        grid=(_B, _NB),
        in_specs=[
            pl.BlockSpec(memory_space=pltpu.SMEM),
            pl.BlockSpec((1, 9, _FB), lambda b, i: (b, 0, i)),
            pl.BlockSpec((128, 2176), lambda b, i: (0, 0)),
            pl.BlockSpec((1, 1, 128), lambda b, i: (b, 0, 0)),
        ],
        out_specs=pl.BlockSpec((1, _FB, 128), lambda b, i: (b, i, 0)),
        out_shape=jax.ShapeDtypeStruct((_B, _NFP, 128), jnp.float32),
        scratch_shapes=[pltpu.VMEM((2176, _FB), jnp.bfloat16)],
    )(vecn, fc, tt, bias3)


def kernel(vertices, faces, in_em, coor_embed, angle_embed, area_embed,
           normal_embed, emnoangle_embed, emangle_embed, emfreq_embed,
           W_proj, b_proj):
    tr = jnp.deg2rad(in_em[:, 0])
    pr = jnp.deg2rad(in_em[:, 1])
    vec = jnp.stack([jnp.sin(pr) * jnp.cos(tr), jnp.sin(pr) * jnp.sin(tr),
                     jnp.cos(pr)], axis=1)
    vn = jnp.linalg.norm(vec, axis=-1, keepdims=True)
    vecn = vec / jnp.maximum(vn, 1e-12)
    abins = _dis(vec, -1.0, 1.0, 128)
    fbin = _dis(in_em[:, 2], 0.0, 1.0, 512)
    em_bins = jnp.concatenate([abins, fbin[:, None]], axis=1)
    vecn_pad = jnp.pad(vecn, ((0, 0), (0, 1)))

    tt, bias8 = _prep_call(
        (coor_embed, angle_embed, area_embed, normal_embed, emnoangle_embed,
         emangle_embed, emfreq_embed),
        W_proj, jnp.reshape(b_proj, (1, 128)), em_bins)

    vt = vertices.reshape(_B * _NV * 3)
    faces_flat = jnp.pad(faces, ((0, 0), (0, _NFP - _NF), (0, 0))).reshape(-1)
    fc = _sc_gather(vt, faces_flat.astype(jnp.int32))

    bias3 = bias8[:_B][:, None, :]
    out = _main_call(vecn_pad.astype(jnp.float32), fc, tt, bias3)
    return out[:, :_NF, :]


# speedup vs baseline: 1.4673x; 1.4673x over previous
import functools

import numpy as np
import jax
import jax.numpy as jnp
from jax import lax
from jax.experimental import pallas as pl
from jax.experimental.pallas import tpu as pltpu
from jax.experimental.pallas import tpu_sc as plsc

_B, _NV, _NF = 2, 20000, 25000
_NFP = 25088
_NW = 32
_FPW = _NFP // _NW
_FB = 512
_NB = _NFP // _FB
_PI = float(np.pi)

_FEATS = ([(64 * k, 64) for k in range(9)]
          + [(576 + 16 * a, 16) for a in range(3)]
          + [(624, 16)]
          + [(640 + 64 * a, 64) for a in range(3)]
          + [(832, 16)])
_EMANGLE_OFF = 848
_EMFREQ_OFF = 1040


def _dis(t, lo, hi, num):
    t2 = (t - lo) / (hi - lo) * num - 0.5
    return jnp.clip(jnp.round(t2), 0, num - 1).astype(jnp.int32)


@functools.cache
def _make_sc_gather():
    mesh = plsc.VectorSubcoreMesh(core_axis_name="c", subcore_axis_name="s")

    @functools.partial(
        pl.kernel,
        out_type=jax.ShapeDtypeStruct((_B * 9 * _NFP,), jnp.float32),
        scratch_types=[
            pltpu.VMEM((3 * _NV,), jnp.float32),
            pltpu.VMEM((3 * _FPW,), jnp.int32),
            pltpu.VMEM((9 * _FPW,), jnp.float32),
        ],
        compiler_params=pltpu.CompilerParams(needs_layout_passes=False),
        mesh=mesh,
    )
    def _sc_body(vt_hbm, faces_hbm, fc_hbm, vbuf, ibuf, obuf):
        wid = lax.axis_index("s") * 2 + lax.axis_index("c")
        base = wid * _FPW
        for b in range(_B):
            pltpu.sync_copy(vt_hbm.at[pl.ds(b * 3 * _NV, 3 * _NV)], vbuf)
            for s in range(3):
                pltpu.sync_copy(
                    faces_hbm.at[pl.ds((b * 3 + s) * _NFP + base, _FPW)],
                    ibuf.at[pl.ds(s * _FPW, _FPW)])

            def body(g, carry):
                for s in range(3):
                    vidx = ibuf[pl.ds(s * _FPW + g * 16, 16)]
                    for c in range(3):
                        vals = plsc.load_gather(vbuf, [vidx + c * _NV])
                        obuf[pl.ds((3 * s + c) * _FPW + g * 16, 16)] = vals
                return carry

            lax.fori_loop(0, _FPW // 16, body, 0)
            for p in range(9):
                pltpu.sync_copy(obuf.at[pl.ds(p * _FPW, _FPW)],
                                fc_hbm.at[pl.ds((b * 9 + p) * _NFP + base, _FPW)])

    return _sc_body


def _sc_gather(vt, faces_t):
    fc_flat = _make_sc_gather()(vt.reshape(-1), faces_t.reshape(-1))
    return fc_flat.reshape(_B, 9, _NFP)


def _prep_body(coor_ref, angle_ref, area_ref, normal_ref, emno_ref,
               emang_ref, emfreq_ref, w_ref, bproj_ref, bins_ref,
               tt_ref, bias_ref):
    embs = [coor_ref] * 9 + [angle_ref] * 3 + [area_ref] + [normal_ref] * 3 + [emno_ref]
    for j, (off, c) in enumerate(_FEATS):
        w = w_ref[off:off + c, :]
        e = embs[j][:, :]
        tjt = lax.dot_general(w, e, (((0,), (1,)), ((), ())),
                              preferred_element_type=jnp.float32)
        tt_ref[:, 128 * j:128 * (j + 1)] = tjt.astype(jnp.bfloat16)

    bias = jnp.broadcast_to(bproj_ref[:, :], (8, 128))
    si = lax.broadcasted_iota(jnp.int32, (8, 128), 0)
    li = lax.broadcasted_iota(jnp.int32, (8, 128), 1)
    for a in range(3):
        c0 = bins_ref[0, a]
        c1 = bins_ref[1, a]
        tgt = jnp.where(si == 0, c0, jnp.where(si == 1, c1, -1))
        oh = (li == tgt).astype(jnp.float32)
        wa = w_ref[_EMANGLE_OFF + 64 * a:_EMANGLE_OFF + 64 * a + 64, :]
        embw = lax.dot_general(emang_ref[:, :], wa, (((1,), (0,)), ((), ())),
                               preferred_element_type=jnp.float32)
        bias = bias + lax.dot_general(oh, embw, (((1,), (0,)), ((), ())),
                                      preferred_element_type=jnp.float32)
    sif = lax.broadcasted_iota(jnp.int32, (8, 512), 0)
    lif = lax.broadcasted_iota(jnp.int32, (8, 512), 1)
    f0 = bins_ref[0, 3]
    f1 = bins_ref[1, 3]
    tgtf = jnp.where(sif == 0, f0, jnp.where(sif == 1, f1, -1))
    ohf = (lif == tgtf).astype(jnp.float32)
    embwf = lax.dot_general(emfreq_ref[:, :], w_ref[_EMFREQ_OFF:, :],
                            (((1,), (0,)), ((), ())),
                            preferred_element_type=jnp.float32)
    bias = bias + lax.dot_general(ohf, embwf, (((1,), (0,)), ((), ())),
                                  preferred_element_type=jnp.float32)
    bias_ref[:, :] = bias


def _prep_call(embeds, w, bprojT, em_bins):
    n_in = len(embeds) + 2
    in_specs = [pl.BlockSpec(x.shape, lambda i, _r=x.ndim: (0,) * _r)
                for x in (*embeds, w, bprojT)]
    in_specs.append(pl.BlockSpec(memory_space=pltpu.SMEM))
    return pl.pallas_call(
        _prep_body,
        grid=(1,),
        in_specs=in_specs,
        out_specs=[pl.BlockSpec((128, 2176), lambda i: (0, 0)),
                   pl.BlockSpec((8, 128), lambda i: (0, 0))],
        out_shape=[jax.ShapeDtypeStruct((128, 2176), jnp.bfloat16),
                   jax.ShapeDtypeStruct((8, 128), jnp.float32)],
    )(*embeds, w, bprojT, em_bins)


def _acos(x):
    ax = jnp.abs(x)
    p = jnp.float32(-0.0012624911)
    for coef in (0.0066700901, -0.0170881256, 0.0308918810, -0.0501743046,
                 0.0889789874, -0.2145988016, 1.5707963050):
        p = p * ax + jnp.float32(coef)
    r = jnp.sqrt(jnp.maximum(1.0 - ax, 0.0)) * p
    return jnp.where(x < 0, _PI - r, r)


def _main_body(vecn_ref, fc_ref, tt_ref, bias_ref, out_ref, oh_ref):
    b = pl.program_id(0)
    v = [[fc_ref[0, 3 * s + c:3 * s + c + 1, :] for c in range(3)] for s in range(3)]
    e = [[v[s][c] - v[s - 1][c] for c in range(3)] for s in range(3)]
    nh = []
    for s in range(3):
        n2 = e[s][0] * e[s][0] + e[s][1] * e[s][1] + e[s][2] * e[s][2]
        den = jnp.maximum(jnp.sqrt(n2), 1e-12)
        nh.append([e[s][c] / den for c in range(3)])

    idxs = []
    for s in range(3):
        for c in range(3):
            idxs.append(_dis(v[s][c], -1.0, 1.0, 128))
    for c in range(3):
        nd = -(nh[0][c] * nh[0][c - 1] + nh[1][c] * nh[1][c - 1]
               + nh[2][c] * nh[2][c - 1])
        ang = _acos(jnp.clip(nd, -1.0 + 1e-5, 1.0 - 1e-5))
        idxs.append(_dis(ang, 0.0, _PI, 128))
    crx = e[0][1] * e[1][2] - e[0][2] * e[1][1]
    cry = e[0][2] * e[1][0] - e[0][0] * e[1][2]
    crz = e[0][0] * e[1][1] - e[0][1] * e[1][0]
    crn = jnp.sqrt(crx * crx + cry * cry + crz * crz)
    dn = jnp.maximum(crn, 1e-12)
    nr = [crx / dn, cry / dn, crz / dn]
    idxs.append(_dis(crn * 0.5, 0.0, 4.0, 128))
    for c in range(3):
        idxs.append(_dis(nr[c], -1.0, 1.0, 128))
    nn = jnp.sqrt(nr[0] * nr[0] + nr[1] * nr[1] + nr[2] * nr[2])
    dnn = jnp.maximum(nn, 1e-12)
    vx = vecn_ref[b, 0]
    vy = vecn_ref[b, 1]
    vz = vecn_ref[b, 2]
    nd2 = -((nr[0] / dnn) * vx + (nr[1] / dnn) * vy + (nr[2] / dnn) * vz)
    emno = _acos(jnp.clip(nd2, -1.0 + 1e-5, 1.0 - 1e-5))
    idxs.append(_dis(emno, 0.0, _PI, 128))

    riota = lax.broadcasted_iota(jnp.int32, (128, _FB), 0)
    for j in range(17):
        oh_ref[128 * j:128 * (j + 1), :] = (riota == idxs[j]).astype(jnp.bfloat16)
    acc = lax.dot_general(oh_ref[:, :], tt_ref[:, :], (((0,), (1,)), ((), ())),
                          preferred_element_type=jnp.float32)
    out_ref[0, :, :] = acc + bias_ref[0, :, :]


def _main_call(vecn, fc, tt, bias3):
    return pl.pallas_call(
        _main_body,
        grid=(_B, _NB),
        in_specs=[
            pl.BlockSpec(memory_space=pltpu.SMEM),
            pl.BlockSpec((1, 9, _FB), lambda b, i: (b, 0, i)),
            pl.BlockSpec((128, 2176), lambda b, i: (0, 0)),
            pl.BlockSpec((1, 1, 128), lambda b, i: (b, 0, 0)),
        ],
        out_specs=pl.BlockSpec((1, _FB, 128), lambda b, i: (b, i, 0)),
        out_shape=jax.ShapeDtypeStruct((_B, _NFP, 128), jnp.float32),
        scratch_shapes=[pltpu.VMEM((2176, _FB), jnp.bfloat16)],
    )(vecn, fc, tt, bias3)


def kernel(vertices, faces, in_em, coor_embed, angle_embed, area_embed,
           normal_embed, emnoangle_embed, emangle_embed, emfreq_embed,
           W_proj, b_proj):
    tr = jnp.deg2rad(in_em[:, 0])
    pr = jnp.deg2rad(in_em[:, 1])
    vec = jnp.stack([jnp.sin(pr) * jnp.cos(tr), jnp.sin(pr) * jnp.sin(tr),
                     jnp.cos(pr)], axis=1)
    vn = jnp.linalg.norm(vec, axis=-1, keepdims=True)
    vecn = vec / jnp.maximum(vn, 1e-12)
    abins = _dis(vec, -1.0, 1.0, 128)
    fbin = _dis(in_em[:, 2], 0.0, 1.0, 512)
    em_bins = jnp.concatenate([abins, fbin[:, None]], axis=1)
    vecn_pad = jnp.pad(vecn, ((0, 0), (0, 1)))

    tt, bias8 = _prep_call(
        (coor_embed, angle_embed, area_embed, normal_embed, emnoangle_embed,
         emangle_embed, emfreq_embed),
        W_proj, jnp.reshape(b_proj, (1, 128)), em_bins)

    vt = vertices.transpose(0, 2, 1).reshape(_B, 3 * _NV)
    faces_t = jnp.pad(faces, ((0, 0), (0, _NFP - _NF), (0, 0))).transpose(0, 2, 1)
    fc = _sc_gather(vt, faces_t.astype(jnp.int32))

    bias3 = bias8[:_B][:, None, :]
    out = _main_call(vecn_pad.astype(jnp.float32), fc, tt, bias3)
    return out[:, :_NF, :]


# speedup vs baseline: 1.6450x; 1.1211x over previous
import functools

import numpy as np
import jax
import jax.numpy as jnp
from jax import lax
from jax.experimental import pallas as pl
from jax.experimental.pallas import tpu as pltpu
from jax.experimental.pallas import tpu_sc as plsc

_B, _NV, _NF = 2, 20000, 25000
_NFP = 25088
_NW = 32
_FPW = _NFP // _NW
_FB = 1024
_NB = _NFP // _FB
_PI = float(np.pi)

_FEATS = ([(64 * k, 64) for k in range(9)]
          + [(576 + 16 * a, 16) for a in range(3)]
          + [(624, 16)]
          + [(640 + 64 * a, 64) for a in range(3)]
          + [(832, 16)])
_EMANGLE_OFF = 848
_EMFREQ_OFF = 1040


def _dis(t, lo, hi, num):
    t2 = (t - lo) / (hi - lo) * num - 0.5
    return jnp.clip(jnp.round(t2), 0, num - 1).astype(jnp.int32)


@functools.cache
def _make_sc_gather():
    mesh = plsc.VectorSubcoreMesh(core_axis_name="c", subcore_axis_name="s")

    @functools.partial(
        pl.kernel,
        out_type=jax.ShapeDtypeStruct((_B * 9 * _NFP,), jnp.float32),
        scratch_types=[
            pltpu.VMEM((3 * _NV,), jnp.float32),
            pltpu.VMEM((3 * _FPW,), jnp.int32),
            pltpu.VMEM((9 * _FPW,), jnp.float32),
        ],
        compiler_params=pltpu.CompilerParams(needs_layout_passes=False),
        mesh=mesh,
    )
    def _sc_body(vt_hbm, faces_hbm, fc_hbm, vbuf, ibuf, obuf):
        wid = lax.axis_index("s") * 2 + lax.axis_index("c")
        base = wid * _FPW
        for b in range(_B):
            pltpu.sync_copy(vt_hbm.at[pl.ds(b * 3 * _NV, 3 * _NV)], vbuf)
            for s in range(3):
                pltpu.sync_copy(
                    faces_hbm.at[pl.ds((b * 3 + s) * _NFP + base, _FPW)],
                    ibuf.at[pl.ds(s * _FPW, _FPW)])

            def body(g, carry):
                for s in range(3):
                    vidx = ibuf[pl.ds(s * _FPW + g * 16, 16)]
                    for c in range(3):
                        vals = plsc.load_gather(vbuf, [vidx + c * _NV])
                        obuf[pl.ds((3 * s + c) * _FPW + g * 16, 16)] = vals
                return carry

            lax.fori_loop(0, _FPW // 16, body, 0)
            for p in range(9):
                pltpu.sync_copy(obuf.at[pl.ds(p * _FPW, _FPW)],
                                fc_hbm.at[pl.ds((b * 9 + p) * _NFP + base, _FPW)])

    return _sc_body


def _sc_gather(vt, faces_t):
    fc_flat = _make_sc_gather()(vt.reshape(-1), faces_t.reshape(-1))
    return fc_flat.reshape(_B, 9, _NFP)


def _prep_body(coor_ref, angle_ref, area_ref, normal_ref, emno_ref,
               emang_ref, emfreq_ref, w_ref, bproj_ref, bins_ref,
               tt_ref, bias_ref):
    embs = [coor_ref] * 9 + [angle_ref] * 3 + [area_ref] + [normal_ref] * 3 + [emno_ref]
    for j, (off, c) in enumerate(_FEATS):
        w = w_ref[off:off + c, :]
        e = embs[j][:, :]
        tjt = lax.dot_general(w, e, (((0,), (1,)), ((), ())),
                              preferred_element_type=jnp.float32)
        tt_ref[:, 128 * j:128 * (j + 1)] = tjt.astype(jnp.bfloat16)

    bias = jnp.broadcast_to(bproj_ref[:, :], (8, 128))
    si = lax.broadcasted_iota(jnp.int32, (8, 128), 0)
    li = lax.broadcasted_iota(jnp.int32, (8, 128), 1)
    for a in range(3):
        c0 = bins_ref[0, a]
        c1 = bins_ref[1, a]
        tgt = jnp.where(si == 0, c0, jnp.where(si == 1, c1, -1))
        oh = (li == tgt).astype(jnp.float32)
        wa = w_ref[_EMANGLE_OFF + 64 * a:_EMANGLE_OFF + 64 * a + 64, :]
        embw = lax.dot_general(emang_ref[:, :], wa, (((1,), (0,)), ((), ())),
                               preferred_element_type=jnp.float32)
        bias = bias + lax.dot_general(oh, embw, (((1,), (0,)), ((), ())),
                                      preferred_element_type=jnp.float32)
    sif = lax.broadcasted_iota(jnp.int32, (8, 512), 0)
    lif = lax.broadcasted_iota(jnp.int32, (8, 512), 1)
    f0 = bins_ref[0, 3]
    f1 = bins_ref[1, 3]
    tgtf = jnp.where(sif == 0, f0, jnp.where(sif == 1, f1, -1))
    ohf = (lif == tgtf).astype(jnp.float32)
    embwf = lax.dot_general(emfreq_ref[:, :], w_ref[_EMFREQ_OFF:, :],
                            (((1,), (0,)), ((), ())),
                            preferred_element_type=jnp.float32)
    bias = bias + lax.dot_general(ohf, embwf, (((1,), (0,)), ((), ())),
                                  preferred_element_type=jnp.float32)
    bias_ref[:, :] = bias


def _prep_call(embeds, w, bprojT, em_bins):
    n_in = len(embeds) + 2
    in_specs = [pl.BlockSpec(x.shape, lambda i, _r=x.ndim: (0,) * _r)
                for x in (*embeds, w, bprojT)]
    in_specs.append(pl.BlockSpec(memory_space=pltpu.SMEM))
    return pl.pallas_call(
        _prep_body,
        grid=(1,),
        in_specs=in_specs,
        out_specs=[pl.BlockSpec((128, 2176), lambda i: (0, 0)),
                   pl.BlockSpec((8, 128), lambda i: (0, 0))],
        out_shape=[jax.ShapeDtypeStruct((128, 2176), jnp.bfloat16),
                   jax.ShapeDtypeStruct((8, 128), jnp.float32)],
    )(*embeds, w, bprojT, em_bins)


def _acos(x):
    ax = jnp.abs(x)
    p = jnp.float32(-0.0012624911)
    for coef in (0.0066700901, -0.0170881256, 0.0308918810, -0.0501743046,
                 0.0889789874, -0.2145988016, 1.5707963050):
        p = p * ax + jnp.float32(coef)
    r = jnp.sqrt(jnp.maximum(1.0 - ax, 0.0)) * p
    return jnp.where(x < 0, _PI - r, r)


def _main_body(vecn_ref, fc_ref, tt_ref, bias_ref, out_ref, oh_ref):
    b = pl.program_id(0)
    v = [[fc_ref[0, 3 * s + c:3 * s + c + 1, :] for c in range(3)] for s in range(3)]
    e = [[v[s][c] - v[s - 1][c] for c in range(3)] for s in range(3)]
    nh = []
    for s in range(3):
        n2 = e[s][0] * e[s][0] + e[s][1] * e[s][1] + e[s][2] * e[s][2]
        den = jnp.maximum(jnp.sqrt(n2), 1e-12)
        nh.append([e[s][c] / den for c in range(3)])

    idxs = []
    for s in range(3):
        for c in range(3):
            idxs.append(_dis(v[s][c], -1.0, 1.0, 128))
    for c in range(3):
        nd = -(nh[0][c] * nh[0][c - 1] + nh[1][c] * nh[1][c - 1]
               + nh[2][c] * nh[2][c - 1])
        ang = _acos(jnp.clip(nd, -1.0 + 1e-5, 1.0 - 1e-5))
        idxs.append(_dis(ang, 0.0, _PI, 128))
    crx = e[0][1] * e[1][2] - e[0][2] * e[1][1]
    cry = e[0][2] * e[1][0] - e[0][0] * e[1][2]
    crz = e[0][0] * e[1][1] - e[0][1] * e[1][0]
    crn = jnp.sqrt(crx * crx + cry * cry + crz * crz)
    dn = jnp.maximum(crn, 1e-12)
    nr = [crx / dn, cry / dn, crz / dn]
    idxs.append(_dis(crn * 0.5, 0.0, 4.0, 128))
    for c in range(3):
        idxs.append(_dis(nr[c], -1.0, 1.0, 128))
    nn = jnp.sqrt(nr[0] * nr[0] + nr[1] * nr[1] + nr[2] * nr[2])
    dnn = jnp.maximum(nn, 1e-12)
    vx = vecn_ref[b, 0]
    vy = vecn_ref[b, 1]
    vz = vecn_ref[b, 2]
    nd2 = -((nr[0] / dnn) * vx + (nr[1] / dnn) * vy + (nr[2] / dnn) * vz)
    emno = _acos(jnp.clip(nd2, -1.0 + 1e-5, 1.0 - 1e-5))
    idxs.append(_dis(emno, 0.0, _PI, 128))

    riota = lax.broadcasted_iota(jnp.int32, (128, _FB), 0)
    for j in range(17):
        oh_ref[128 * j:128 * (j + 1), :] = (riota == idxs[j]).astype(jnp.bfloat16)
    acc = lax.dot_general(oh_ref[:, :], tt_ref[:, :], (((0,), (1,)), ((), ())),
                          preferred_element_type=jnp.float32)
    out_ref[0, :, :] = acc + bias_ref[0, :, :]


def _main_call(vecn, fc, tt, bias3):
    return pl.pallas_call(
        _main_body,
        grid=(_B, _NB),
        in_specs=[
            pl.BlockSpec(memory_space=pltpu.SMEM),
            pl.BlockSpec((1, 9, _FB), lambda b, i: (b, 0, i)),
            pl.BlockSpec((128, 2176), lambda b, i: (0, 0)),
            pl.BlockSpec((1, 1, 128), lambda b, i: (b, 0, 0)),
        ],
        out_specs=pl.BlockSpec((1, _FB, 128), lambda b, i: (b, i, 0)),
        out_shape=jax.ShapeDtypeStruct((_B, _NFP, 128), jnp.float32),
        scratch_shapes=[pltpu.VMEM((2176, _FB), jnp.bfloat16)],
    )(vecn, fc, tt, bias3)


def kernel(vertices, faces, in_em, coor_embed, angle_embed, area_embed,
           normal_embed, emnoangle_embed, emangle_embed, emfreq_embed,
           W_proj, b_proj):
    tr = jnp.deg2rad(in_em[:, 0])
    pr = jnp.deg2rad(in_em[:, 1])
    vec = jnp.stack([jnp.sin(pr) * jnp.cos(tr), jnp.sin(pr) * jnp.sin(tr),
                     jnp.cos(pr)], axis=1)
    vn = jnp.linalg.norm(vec, axis=-1, keepdims=True)
    vecn = vec / jnp.maximum(vn, 1e-12)
    abins = _dis(vec, -1.0, 1.0, 128)
    fbin = _dis(in_em[:, 2], 0.0, 1.0, 512)
    em_bins = jnp.concatenate([abins, fbin[:, None]], axis=1)
    vecn_pad = jnp.pad(vecn, ((0, 0), (0, 1)))

    tt, bias8 = _prep_call(
        (coor_embed, angle_embed, area_embed, normal_embed, emnoangle_embed,
         emangle_embed, emfreq_embed),
        W_proj, jnp.reshape(b_proj, (1, 128)), em_bins)

    vt = vertices.transpose(0, 2, 1).reshape(_B, 3 * _NV)
    faces_t = jnp.pad(faces, ((0, 0), (0, _NFP - _NF), (0, 0))).transpose(0, 2, 1)
    fc = _sc_gather(vt, faces_t.astype(jnp.int32))

    bias3 = bias8[:_B][:, None, :]
    out = _main_call(vecn_pad.astype(jnp.float32), fc, tt, bias3)
    return out[:, :_NF, :]
